# baseline (speedup 1.0000x reference)
import functools

import jax
import jax.numpy as jnp
from jax.experimental import pallas as pl
from jax.experimental.pallas import tpu as pltpu

_PREC = jax.lax.Precision.HIGHEST
_F32 = jnp.float32


def _row3(x_ref, y, cin, wp):
    return x_ref[pl.ds(y, 3)].reshape(3 * cin, wp)


def _tap_matmul(wfun, x3, w):
    acc = jnp.dot(wfun(0), x3[:, 0:w], precision=_PREC)
    acc = acc + jnp.dot(wfun(1), x3[:, 1:w + 1], precision=_PREC)
    acc = acc + jnp.dot(wfun(2), x3[:, 2:w + 2], precision=_PREC)
    return acc


def _pad_row(acc):
    cout = acc.shape[0]
    zcol = jnp.zeros((cout, 1), _F32)
    return jnp.concatenate([zcol, acc, zcol], axis=1)


def _conv_kernel(x_ref, w_ref, o_ref, *, hh, ww, relu, pad_out):
    cin = x_ref.shape[1]
    cout = o_ref.shape[1]
    if pad_out:
        o_ref[0] = jnp.zeros((cout, ww + 2), _F32)
        o_ref[hh + 1] = jnp.zeros((cout, ww + 2), _F32)

    def row(y, carry):
        x3 = _row3(x_ref, y, cin, ww + 2)
        acc = _tap_matmul(lambda dx: w_ref[dx], x3, ww)
        if relu:
            acc = jnp.maximum(acc, 0.0)
        if pad_out:
            o_ref[y + 1] = _pad_row(acc)
        else:
            o_ref[y] = acc
        return carry

    jax.lax.fori_loop(0, hh, row, 0)


def _conv3x3(xp, w3, relu, pad_out):
    hp, _, wp = xp.shape
    hh, ww = hp - 2, wp - 2
    cout = w3.shape[1]
    oshape = (hp, cout, wp) if pad_out else (hh, cout, ww)
    return pl.pallas_call(
        functools.partial(_conv_kernel, hh=hh, ww=ww, relu=relu,
                          pad_out=pad_out),
        out_shape=jax.ShapeDtypeStruct(oshape, _F32),
    )(xp, w3)


def _musig_kernel(h_ref, wms_ref, mu_ref, sg_ref, *, hh, ww):
    c = mu_ref.shape[1]

    def row(y, carry):
        hrow = h_ref[y + 1, :, 1:ww + 1]
        ms = jnp.dot(wms_ref[...], hrow, precision=_PREC)
        mu_ref[y] = ms[:c]
        raw = ms[c:]
        sp = jnp.maximum(raw, 0.0) + jnp.log1p(jnp.exp(-jnp.abs(raw)))
        sg_ref[y] = sp + 1e-6
        return carry

    jax.lax.fori_loop(0, hh, row, 0)


def _musig(hp, wms):
    hp_, c2 = hp.shape[0], wms.shape[0]
    hh, ww = hp_ - 2, hp.shape[2] - 2
    c = c2 // 2
    return pl.pallas_call(
        functools.partial(_musig_kernel, hh=hh, ww=ww),
        out_shape=(jax.ShapeDtypeStruct((hh, c, ww), _F32),
                   jax.ShapeDtypeStruct((hh, c, ww), _F32)),
    )(hp, wms)


def _viz_kernel(mu_ref, sg_ref, eps_ref, wr_ref, bm_ref, z_ref, rt_ref,
                acc_ref, *, hh, ww):
    c = mu_ref.shape[1]
    z_ref[0] = jnp.zeros((c, ww + 2), _F32)
    z_ref[hh + 1] = jnp.zeros((c, ww + 2), _F32)
    acc_ref[...] = jnp.zeros((c, ww), _F32)

    def row(y, carry):
        z = mu_ref[y] + jnp.sqrt(sg_ref[y]) * eps_ref[y]
        z_ref[y + 1] = _pad_row(z)
        acc_ref[...] += z
        return carry

    jax.lax.fori_loop(0, hh, row, 0)

    pooled_mat = jnp.dot(wr_ref[...], acc_ref[...], precision=_PREC)
    logits = jnp.sum(pooled_mat, axis=1, keepdims=True) / (hh * ww)
    logits = logits + bm_ref[...]
    sub = jax.lax.broadcasted_iota(jnp.int32, (8, 1), 0)
    neg = jnp.float32(-1e30)
    m1 = jnp.max(logits)
    i1 = -jnp.max(jnp.where(logits == m1, -sub.astype(_F32), neg))
    masked = jnp.where(sub.astype(_F32) == i1, neg, logits)
    m2 = jnp.max(masked)
    i2 = -jnp.max(jnp.where(masked == m2, -sub.astype(_F32), neg))
    e1 = jnp.exp(m1 - m1)
    e2 = jnp.exp(m2 - m1)
    g1 = e1 / (e1 + e2)
    g2 = e2 / (e1 + e2)
    out = jnp.where(sub == 0, i1,
          jnp.where(sub == 1, i2,
          jnp.where(sub == 2, g1,
          jnp.where(sub == 3, g2, 0.0))))
    rt_ref[...] = jnp.broadcast_to(out, (8, 128))


def _viz(mu, sg, eps, wr, bmask):
    hh, c, ww = mu.shape
    return pl.pallas_call(
        functools.partial(_viz_kernel, hh=hh, ww=ww),
        out_shape=(jax.ShapeDtypeStruct((hh + 2, c, ww + 2), _F32),
                   jax.ShapeDtypeStruct((8, 128), _F32)),
        scratch_shapes=[pltpu.VMEM((c, ww), _F32)],
    )(mu, sg, eps, wr, bmask)


def _moe_kernel(z_ref, w1_ref, w2_ref, idx_ref, gate_ref, o_ref, a_ref,
                *, hh, ww):
    c = z_ref.shape[1]
    cout = o_ref.shape[1]
    o_ref[0] = jnp.zeros((cout, ww + 2), _F32)
    o_ref[hh + 1] = jnp.zeros((cout, ww + 2), _F32)
    a_ref[0] = jnp.zeros((c, ww + 2), _F32)
    a_ref[hh + 1] = jnp.zeros((c, ww + 2), _F32)

    for k in (0, 1):
        e = idx_ref[k]
        g = gate_ref[k]

        def row1(y, carry):
            x3 = _row3(z_ref, y, c, ww + 2)
            acc = _tap_matmul(lambda dx: w1_ref[e, dx], x3, ww)
            a_ref[y + 1] = _pad_row(jnp.maximum(acc, 0.0))
            return carry

        jax.lax.fori_loop(0, hh, row1, 0)

        def row2(y, carry):
            x3 = _row3(a_ref, y, c, ww + 2)
            acc = _tap_matmul(lambda dx: w2_ref[e, dx], x3, ww) * g
            if k == 0:
                o_ref[y + 1] = _pad_row(acc)
            else:
                o_ref[y + 1] = o_ref[y + 1] + _pad_row(acc)
            return carry

        jax.lax.fori_loop(0, hh, row2, 0)


def _moe(zp, w1, w2, idx, gate):
    hp, c, wp = zp.shape
    hh, ww = hp - 2, wp - 2
    cout = w2.shape[2]
    return pl.pallas_call(
        functools.partial(_moe_kernel, hh=hh, ww=ww),
        out_shape=jax.ShapeDtypeStruct((hp, cout, wp), _F32),
        in_specs=[pl.BlockSpec(memory_space=pltpu.VMEM),
                  pl.BlockSpec(memory_space=pltpu.VMEM),
                  pl.BlockSpec(memory_space=pltpu.VMEM),
                  pl.BlockSpec(memory_space=pltpu.SMEM),
                  pl.BlockSpec(memory_space=pltpu.SMEM)],
        scratch_shapes=[pltpu.VMEM((hp, c, wp), _F32)],
    )(zp, w1, w2, idx, gate)


def _add4_kernel(v_ref, g_ref, i_ref, l_ref, o_ref, *, hp):
    cpad = o_ref.shape[1] - v_ref.shape[1]
    wp = v_ref.shape[2]

    def row(y, carry):
        s = v_ref[y] + g_ref[y] + (i_ref[y] + l_ref[y])
        o_ref[y] = jnp.concatenate([s, jnp.zeros((cpad, wp), _F32)], axis=0)
        return carry

    jax.lax.fori_loop(0, hp, row, 0)


def _add4(vp, gp, ip, lp):
    hp, _, wp = vp.shape
    return pl.pallas_call(
        functools.partial(_add4_kernel, hp=hp),
        out_shape=jax.ShapeDtypeStruct((hp, 8, wp), _F32),
    )(vp, gp, ip, lp)


def _prep_w3(w, cin_pad=None):
    cout, cin = w.shape[0], w.shape[1]
    if cin_pad is not None and cin_pad > cin:
        w = jnp.pad(w, ((0, 0), (0, cin_pad - cin), (0, 0), (0, 0)))
        cin = cin_pad
    return jnp.transpose(w, (3, 0, 2, 1)).reshape(3, cout, 3 * cin)


def _prep_wexp(w):
    e, cout, cin = w.shape[0], w.shape[1], w.shape[2]
    return jnp.transpose(w, (0, 4, 1, 3, 2)).reshape(e, 3, cout, 3 * cin)


def _to_hcw_pad(x, cpad=None):
    t = jnp.transpose(x[0], (1, 0, 2))
    extra = 0 if cpad is None else max(0, cpad - t.shape[1])
    return jnp.pad(t, ((1, 1), (0, extra), (1, 1)))


def _to_hcw(x):
    return jnp.transpose(x[0], (1, 0, 2))


def _from_hcw(x):
    return jnp.transpose(x, (1, 0, 2))[None]


def _unpad(xp):
    return xp[1:-1, :, 1:-1]


def _branch(x_nchw, eps_nchw, w_stem, w_blocks, w_mu, w_sig, w_router,
            b_router, w_exp1, w_exp2):
    xp = _to_hcw_pad(x_nchw, cpad=8)
    h = _conv3x3(xp, _prep_w3(w_stem, cin_pad=8), relu=True, pad_out=True)
    for bi in range(w_blocks.shape[0]):
        h = _conv3x3(h, _prep_w3(w_blocks[bi]), relu=True, pad_out=True)

    wms = jnp.concatenate([w_mu[:, :, 0, 0], w_sig[:, :, 0, 0]], axis=0)
    mu, sg2 = _musig(h, wms)

    e = w_router.shape[0]
    wr = jnp.pad(w_router, ((0, 8 - e), (0, 0)))
    bmask = jnp.pad(b_router, (0, 8 - e),
                    constant_values=-1e30).reshape(8, 1).astype(_F32)
    zp, rt = _viz(mu, sg2, _to_hcw(eps_nchw), wr, bmask)

    idx = rt[0:2, 0].astype(jnp.int32)
    gate = rt[2:4, 0]
    dec = _moe(zp, _prep_wexp(w_exp1), _prep_wexp(w_exp2), idx, gate)
    return dec, mu, sg2


def kernel(i, v, eps_i, eps_v, W_ie_stem, W_ie_blocks, W_i_mu, W_i_sig,
           W_i_router, b_i_router, W_i_exp1, W_i_exp2, W_ve_stem, W_ve_blocks,
           W_v_mu, W_v_sig, W_v_router, b_v_router, W_v_exp1, W_v_exp2,
           W_f_stem, W_f_blocks, W_f_out):
    lp, mu_l, sg_l = _branch(i, eps_i, W_ie_stem, W_ie_blocks, W_i_mu, W_i_sig,
                             W_i_router, b_i_router, W_i_exp1, W_i_exp2)
    gp, mu_g, sg_g = _branch(v, eps_v, W_ve_stem, W_ve_blocks, W_v_mu, W_v_sig,
                             W_v_router, b_v_router, W_v_exp1, W_v_exp2)

    fused = _add4(_to_hcw_pad(v), gp, _to_hcw_pad(i), lp)
    fh = _conv3x3(fused, _prep_w3(W_f_stem, cin_pad=8), relu=True,
                  pad_out=True)
    for bi in range(W_f_blocks.shape[0]):
        fh = _conv3x3(fh, _prep_w3(W_f_blocks[bi]), relu=True, pad_out=True)
    fusion = _conv3x3(fh, _prep_w3(W_f_out), relu=False, pad_out=False)

    return (_from_hcw(fusion), _from_hcw(_unpad(lp)), _from_hcw(_unpad(gp)),
            _from_hcw(mu_l), _from_hcw(sg_l), _from_hcw(mu_g), _from_hcw(sg_g))


# speedup vs baseline: 1.6135x; 1.6135x over previous
import functools

import jax
import jax.numpy as jnp
from jax.experimental import pallas as pl
from jax.experimental.pallas import tpu as pltpu

_PREC = jax.lax.Precision.HIGHEST
_CPREC = jax.lax.Precision.DEFAULT
_F32 = jnp.float32


def _row3(x_ref, y, cin, wp):
    return x_ref[pl.ds(y, 3)].reshape(3 * cin, wp)


def _tap_matmul(wfun, x3, w):
    acc = jnp.dot(wfun(0), x3[:, 0:w], precision=_CPREC)
    acc = acc + jnp.dot(wfun(1), x3[:, 1:w + 1], precision=_CPREC)
    acc = acc + jnp.dot(wfun(2), x3[:, 2:w + 2], precision=_CPREC)
    return acc


def _pad_row(acc):
    cout = acc.shape[0]
    zcol = jnp.zeros((cout, 1), _F32)
    return jnp.concatenate([zcol, acc, zcol], axis=1)


def _conv_kernel(x_ref, w_ref, o_ref, *, hh, ww, relu, pad_out):
    cin = x_ref.shape[1]
    cout = o_ref.shape[1]
    if pad_out:
        o_ref[0] = jnp.zeros((cout, ww + 2), _F32)
        o_ref[hh + 1] = jnp.zeros((cout, ww + 2), _F32)

    def row(y, carry):
        x3 = _row3(x_ref, y, cin, ww + 2)
        acc = _tap_matmul(lambda dx: w_ref[dx], x3, ww)
        if relu:
            acc = jnp.maximum(acc, 0.0)
        if pad_out:
            o_ref[y + 1] = _pad_row(acc)
        else:
            o_ref[y] = acc
        return carry

    jax.lax.fori_loop(0, hh, row, 0)


def _conv3x3(xp, w3, relu, pad_out):
    hp, _, wp = xp.shape
    hh, ww = hp - 2, wp - 2
    cout = w3.shape[1]
    oshape = (hp, cout, wp) if pad_out else (hh, cout, ww)
    return pl.pallas_call(
        functools.partial(_conv_kernel, hh=hh, ww=ww, relu=relu,
                          pad_out=pad_out),
        out_shape=jax.ShapeDtypeStruct(oshape, _F32),
    )(xp, w3)


def _musig_kernel(h_ref, wms_ref, mu_ref, sg_ref, *, hh, ww):
    c = mu_ref.shape[1]

    def row(y, carry):
        hrow = h_ref[y + 1, :, 1:ww + 1]
        ms = jnp.dot(wms_ref[...], hrow, precision=_CPREC)
        mu_ref[y] = ms[:c]
        raw = ms[c:]
        sp = jnp.maximum(raw, 0.0) + jnp.log1p(jnp.exp(-jnp.abs(raw)))
        sg_ref[y] = sp + 1e-6
        return carry

    jax.lax.fori_loop(0, hh, row, 0)


def _musig(hp, wms):
    hp_, c2 = hp.shape[0], wms.shape[0]
    hh, ww = hp_ - 2, hp.shape[2] - 2
    c = c2 // 2
    return pl.pallas_call(
        functools.partial(_musig_kernel, hh=hh, ww=ww),
        out_shape=(jax.ShapeDtypeStruct((hh, c, ww), _F32),
                   jax.ShapeDtypeStruct((hh, c, ww), _F32)),
    )(hp, wms)


def _viz_kernel(mu_ref, sg_ref, eps_ref, wr_ref, bm_ref, z_ref, rt_ref,
                acc_ref, *, hh, ww):
    c = mu_ref.shape[1]
    z_ref[0] = jnp.zeros((c, ww + 2), _F32)
    z_ref[hh + 1] = jnp.zeros((c, ww + 2), _F32)
    acc_ref[...] = jnp.zeros((c, ww), _F32)

    def row(y, carry):
        z = mu_ref[y] + jnp.sqrt(sg_ref[y]) * eps_ref[y]
        z_ref[y + 1] = _pad_row(z)
        acc_ref[...] += z
        return carry

    jax.lax.fori_loop(0, hh, row, 0)

    pooled_mat = jnp.dot(wr_ref[...], acc_ref[...], precision=_PREC)
    logits = jnp.sum(pooled_mat, axis=1, keepdims=True) / (hh * ww)
    logits = logits + bm_ref[...]
    sub = jax.lax.broadcasted_iota(jnp.int32, (8, 1), 0)
    neg = jnp.float32(-1e30)
    m1 = jnp.max(logits)
    i1 = -jnp.max(jnp.where(logits == m1, -sub.astype(_F32), neg))
    masked = jnp.where(sub.astype(_F32) == i1, neg, logits)
    m2 = jnp.max(masked)
    i2 = -jnp.max(jnp.where(masked == m2, -sub.astype(_F32), neg))
    e1 = jnp.exp(m1 - m1)
    e2 = jnp.exp(m2 - m1)
    g1 = e1 / (e1 + e2)
    g2 = e2 / (e1 + e2)
    out = jnp.where(sub == 0, i1,
          jnp.where(sub == 1, i2,
          jnp.where(sub == 2, g1,
          jnp.where(sub == 3, g2, 0.0))))
    rt_ref[...] = jnp.broadcast_to(out, (8, 128))


def _viz(mu, sg, eps, wr, bmask):
    hh, c, ww = mu.shape
    return pl.pallas_call(
        functools.partial(_viz_kernel, hh=hh, ww=ww),
        out_shape=(jax.ShapeDtypeStruct((hh + 2, c, ww + 2), _F32),
                   jax.ShapeDtypeStruct((8, 128), _F32)),
        scratch_shapes=[pltpu.VMEM((c, ww), _F32)],
    )(mu, sg, eps, wr, bmask)


def _moe_kernel(z_ref, w1_ref, w2_ref, idx_ref, gate_ref, o_ref, a_ref,
                *, hh, ww):
    c = z_ref.shape[1]
    cout = o_ref.shape[1]
    o_ref[0] = jnp.zeros((cout, ww + 2), _F32)
    o_ref[hh + 1] = jnp.zeros((cout, ww + 2), _F32)
    a_ref[0] = jnp.zeros((c, ww + 2), _F32)
    a_ref[hh + 1] = jnp.zeros((c, ww + 2), _F32)

    for k in (0, 1):
        e = idx_ref[k]
        g = gate_ref[k]

        def row1(y, carry):
            x3 = _row3(z_ref, y, c, ww + 2)
            acc = _tap_matmul(lambda dx: w1_ref[e, dx], x3, ww)
            a_ref[y + 1] = _pad_row(jnp.maximum(acc, 0.0))
            return carry

        jax.lax.fori_loop(0, hh, row1, 0)

        def row2(y, carry):
            x3 = _row3(a_ref, y, c, ww + 2)
            acc = _tap_matmul(lambda dx: w2_ref[e, dx], x3, ww) * g
            if k == 0:
                o_ref[y + 1] = _pad_row(acc)
            else:
                o_ref[y + 1] = o_ref[y + 1] + _pad_row(acc)
            return carry

        jax.lax.fori_loop(0, hh, row2, 0)


def _moe(zp, w1, w2, idx, gate):
    hp, c, wp = zp.shape
    hh, ww = hp - 2, wp - 2
    cout = w2.shape[2]
    return pl.pallas_call(
        functools.partial(_moe_kernel, hh=hh, ww=ww),
        out_shape=jax.ShapeDtypeStruct((hp, cout, wp), _F32),
        in_specs=[pl.BlockSpec(memory_space=pltpu.VMEM),
                  pl.BlockSpec(memory_space=pltpu.VMEM),
                  pl.BlockSpec(memory_space=pltpu.VMEM),
                  pl.BlockSpec(memory_space=pltpu.SMEM),
                  pl.BlockSpec(memory_space=pltpu.SMEM)],
        scratch_shapes=[pltpu.VMEM((hp, c, wp), _F32)],
    )(zp, w1, w2, idx, gate)


def _add4_kernel(v_ref, g_ref, i_ref, l_ref, o_ref, *, hp):
    cpad = o_ref.shape[1] - v_ref.shape[1]
    wp = v_ref.shape[2]

    def row(y, carry):
        s = v_ref[y] + g_ref[y] + (i_ref[y] + l_ref[y])
        o_ref[y] = jnp.concatenate([s, jnp.zeros((cpad, wp), _F32)], axis=0)
        return carry

    jax.lax.fori_loop(0, hp, row, 0)


def _add4(vp, gp, ip, lp):
    hp, _, wp = vp.shape
    return pl.pallas_call(
        functools.partial(_add4_kernel, hp=hp),
        out_shape=jax.ShapeDtypeStruct((hp, 8, wp), _F32),
    )(vp, gp, ip, lp)


def _prep_w3(w, cin_pad=None):
    cout, cin = w.shape[0], w.shape[1]
    if cin_pad is not None and cin_pad > cin:
        w = jnp.pad(w, ((0, 0), (0, cin_pad - cin), (0, 0), (0, 0)))
        cin = cin_pad
    return jnp.transpose(w, (3, 0, 2, 1)).reshape(3, cout, 3 * cin)


def _prep_wexp(w):
    e, cout, cin = w.shape[0], w.shape[1], w.shape[2]
    return jnp.transpose(w, (0, 4, 1, 3, 2)).reshape(e, 3, cout, 3 * cin)


def _to_hcw_pad(x, cpad=None):
    t = jnp.transpose(x[0], (1, 0, 2))
    extra = 0 if cpad is None else max(0, cpad - t.shape[1])
    return jnp.pad(t, ((1, 1), (0, extra), (1, 1)))


def _to_hcw(x):
    return jnp.transpose(x[0], (1, 0, 2))


def _from_hcw(x):
    return jnp.transpose(x, (1, 0, 2))[None]


def _unpad(xp):
    return xp[1:-1, :, 1:-1]


def _branch(x_nchw, eps_nchw, w_stem, w_blocks, w_mu, w_sig, w_router,
            b_router, w_exp1, w_exp2):
    xp = _to_hcw_pad(x_nchw, cpad=8)
    h = _conv3x3(xp, _prep_w3(w_stem, cin_pad=8), relu=True, pad_out=True)
    for bi in range(w_blocks.shape[0]):
        h = _conv3x3(h, _prep_w3(w_blocks[bi]), relu=True, pad_out=True)

    wms = jnp.concatenate([w_mu[:, :, 0, 0], w_sig[:, :, 0, 0]], axis=0)
    mu, sg2 = _musig(h, wms)

    e = w_router.shape[0]
    wr = jnp.pad(w_router, ((0, 8 - e), (0, 0)))
    bmask = jnp.pad(b_router, (0, 8 - e),
                    constant_values=-1e30).reshape(8, 1).astype(_F32)
    zp, rt = _viz(mu, sg2, _to_hcw(eps_nchw), wr, bmask)

    idx = rt[0:2, 0].astype(jnp.int32)
    gate = rt[2:4, 0]
    dec = _moe(zp, _prep_wexp(w_exp1), _prep_wexp(w_exp2), idx, gate)
    return dec, mu, sg2


def kernel(i, v, eps_i, eps_v, W_ie_stem, W_ie_blocks, W_i_mu, W_i_sig,
           W_i_router, b_i_router, W_i_exp1, W_i_exp2, W_ve_stem, W_ve_blocks,
           W_v_mu, W_v_sig, W_v_router, b_v_router, W_v_exp1, W_v_exp2,
           W_f_stem, W_f_blocks, W_f_out):
    lp, mu_l, sg_l = _branch(i, eps_i, W_ie_stem, W_ie_blocks, W_i_mu, W_i_sig,
                             W_i_router, b_i_router, W_i_exp1, W_i_exp2)
    gp, mu_g, sg_g = _branch(v, eps_v, W_ve_stem, W_ve_blocks, W_v_mu, W_v_sig,
                             W_v_router, b_v_router, W_v_exp1, W_v_exp2)

    fused = _add4(_to_hcw_pad(v), gp, _to_hcw_pad(i), lp)
    fh = _conv3x3(fused, _prep_w3(W_f_stem, cin_pad=8), relu=True,
                  pad_out=True)
    for bi in range(W_f_blocks.shape[0]):
        fh = _conv3x3(fh, _prep_w3(W_f_blocks[bi]), relu=True, pad_out=True)
    fusion = _conv3x3(fh, _prep_w3(W_f_out), relu=False, pad_out=False)

    return (_from_hcw(fusion), _from_hcw(_unpad(lp)), _from_hcw(_unpad(gp)),
            _from_hcw(mu_l), _from_hcw(sg_l), _from_hcw(mu_g), _from_hcw(sg_g))


# speedup vs baseline: 3.2763x; 2.0305x over previous
import functools

import jax
import jax.numpy as jnp
from jax.experimental import pallas as pl
from jax.experimental.pallas import tpu as pltpu

_PREC = jax.lax.Precision.HIGHEST
_CPREC = jax.lax.Precision.DEFAULT
_F32 = jnp.float32


def _row3(x_ref, y, cin, wp):
    return x_ref[pl.ds(y, 3)].reshape(3 * cin, wp)


def _tap_matmul(wfun, x3, w):
    acc = jnp.dot(wfun(0), x3[:, 0:w], precision=_CPREC)
    acc = acc + jnp.dot(wfun(1), x3[:, 1:w + 1], precision=_CPREC)
    acc = acc + jnp.dot(wfun(2), x3[:, 2:w + 2], precision=_CPREC)
    return acc


def _pad_row(acc):
    cout = acc.shape[0]
    zcol = jnp.zeros((cout, 1), _F32)
    return jnp.concatenate([zcol, acc, zcol], axis=1)


_UNROLL = 4


def _conv_kernel(x_ref, w_ref, o_ref, *, hh, ww, relu, pad_out):
    cin = x_ref.shape[1]
    cout = o_ref.shape[1]
    if pad_out:
        o_ref[0] = jnp.zeros((cout, ww + 2), _F32)
        o_ref[hh + 1] = jnp.zeros((cout, ww + 2), _F32)

    def rows(it, carry):
        y0 = it * _UNROLL
        for u in range(_UNROLL):
            y = y0 + u
            x3 = _row3(x_ref, y, cin, ww + 2)
            acc = _tap_matmul(lambda dx: w_ref[dx], x3, ww)
            if relu:
                acc = jnp.maximum(acc, 0.0)
            if pad_out:
                o_ref[y + 1] = _pad_row(acc)
            else:
                o_ref[y] = acc
        return carry

    jax.lax.fori_loop(0, hh // _UNROLL, rows, 0)


def _conv3x3(xp, w3, relu, pad_out):
    hp, _, wp = xp.shape
    hh, ww = hp - 2, wp - 2
    cout = w3.shape[1]
    oshape = (hp, cout, wp) if pad_out else (hh, cout, ww)
    return pl.pallas_call(
        functools.partial(_conv_kernel, hh=hh, ww=ww, relu=relu,
                          pad_out=pad_out),
        out_shape=jax.ShapeDtypeStruct(oshape, _F32),
    )(xp, w3)


def _musig_kernel(h_ref, wms_ref, mu_ref, sg_ref, *, hh, ww):
    c = mu_ref.shape[1]

    def rows(it, carry):
        y0 = it * _UNROLL
        for u in range(_UNROLL):
            y = y0 + u
            hrow = h_ref[y + 1, :, 1:ww + 1]
            ms = jnp.dot(wms_ref[...], hrow, precision=_CPREC)
            mu_ref[y] = ms[:c]
            raw = ms[c:]
            sp = jnp.maximum(raw, 0.0) + jnp.log1p(jnp.exp(-jnp.abs(raw)))
            sg_ref[y] = sp + 1e-6
        return carry

    jax.lax.fori_loop(0, hh // _UNROLL, rows, 0)


def _musig(hp, wms):
    hp_, c2 = hp.shape[0], wms.shape[0]
    hh, ww = hp_ - 2, hp.shape[2] - 2
    c = c2 // 2
    return pl.pallas_call(
        functools.partial(_musig_kernel, hh=hh, ww=ww),
        out_shape=(jax.ShapeDtypeStruct((hh, c, ww), _F32),
                   jax.ShapeDtypeStruct((hh, c, ww), _F32)),
    )(hp, wms)


def _viz_kernel(mu_ref, sg_ref, eps_ref, wr_ref, bm_ref, z_ref, rt_ref,
                acc_ref, *, hh, ww):
    c = mu_ref.shape[1]
    z_ref[0] = jnp.zeros((c, ww + 2), _F32)
    z_ref[hh + 1] = jnp.zeros((c, ww + 2), _F32)
    acc_ref[...] = jnp.zeros((c, ww), _F32)

    def rows(it, carry):
        y0 = it * _UNROLL
        acc = acc_ref[...]
        for u in range(_UNROLL):
            y = y0 + u
            z = mu_ref[y] + jnp.sqrt(sg_ref[y]) * eps_ref[y]
            z_ref[y + 1] = _pad_row(z)
            acc = acc + z
        acc_ref[...] = acc
        return carry

    jax.lax.fori_loop(0, hh // _UNROLL, rows, 0)

    pooled_mat = jnp.dot(wr_ref[...], acc_ref[...], precision=_PREC)
    logits = jnp.sum(pooled_mat, axis=1, keepdims=True) / (hh * ww)
    logits = logits + bm_ref[...]
    sub = jax.lax.broadcasted_iota(jnp.int32, (8, 1), 0)
    neg = jnp.float32(-1e30)
    m1 = jnp.max(logits)
    i1 = -jnp.max(jnp.where(logits == m1, -sub.astype(_F32), neg))
    masked = jnp.where(sub.astype(_F32) == i1, neg, logits)
    m2 = jnp.max(masked)
    i2 = -jnp.max(jnp.where(masked == m2, -sub.astype(_F32), neg))
    e1 = jnp.exp(m1 - m1)
    e2 = jnp.exp(m2 - m1)
    g1 = e1 / (e1 + e2)
    g2 = e2 / (e1 + e2)
    out = jnp.where(sub == 0, i1,
          jnp.where(sub == 1, i2,
          jnp.where(sub == 2, g1,
          jnp.where(sub == 3, g2, 0.0))))
    rt_ref[...] = jnp.broadcast_to(out, (8, 128))


def _viz(mu, sg, eps, wr, bmask):
    hh, c, ww = mu.shape
    return pl.pallas_call(
        functools.partial(_viz_kernel, hh=hh, ww=ww),
        out_shape=(jax.ShapeDtypeStruct((hh + 2, c, ww + 2), _F32),
                   jax.ShapeDtypeStruct((8, 128), _F32)),
        scratch_shapes=[pltpu.VMEM((c, ww), _F32)],
    )(mu, sg, eps, wr, bmask)


def _moe_kernel(z_ref, w1_ref, w2_ref, idx_ref, gate_ref, o_ref, a_ref,
                *, hh, ww):
    c = z_ref.shape[1]
    cout = o_ref.shape[1]
    o_ref[0] = jnp.zeros((cout, ww + 2), _F32)
    o_ref[hh + 1] = jnp.zeros((cout, ww + 2), _F32)
    a_ref[0] = jnp.zeros((c, ww + 2), _F32)
    a_ref[hh + 1] = jnp.zeros((c, ww + 2), _F32)

    for k in (0, 1):
        e = idx_ref[k]
        g = gate_ref[k]

        def rows1(it, carry):
            y0 = it * _UNROLL
            for u in range(_UNROLL):
                y = y0 + u
                x3 = _row3(z_ref, y, c, ww + 2)
                acc = _tap_matmul(lambda dx: w1_ref[e, dx], x3, ww)
                a_ref[y + 1] = _pad_row(jnp.maximum(acc, 0.0))
            return carry

        jax.lax.fori_loop(0, hh // _UNROLL, rows1, 0)

        def rows2(it, carry):
            y0 = it * _UNROLL
            for u in range(_UNROLL):
                y = y0 + u
                x3 = _row3(a_ref, y, c, ww + 2)
                acc = _tap_matmul(lambda dx: w2_ref[e, dx], x3, ww) * g
                if k == 0:
                    o_ref[y + 1] = _pad_row(acc)
                else:
                    o_ref[y + 1] = o_ref[y + 1] + _pad_row(acc)
            return carry

        jax.lax.fori_loop(0, hh // _UNROLL, rows2, 0)


def _moe(zp, w1, w2, idx, gate):
    hp, c, wp = zp.shape
    hh, ww = hp - 2, wp - 2
    cout = w2.shape[2]
    return pl.pallas_call(
        functools.partial(_moe_kernel, hh=hh, ww=ww),
        out_shape=jax.ShapeDtypeStruct((hp, cout, wp), _F32),
        in_specs=[pl.BlockSpec(memory_space=pltpu.VMEM),
                  pl.BlockSpec(memory_space=pltpu.VMEM),
                  pl.BlockSpec(memory_space=pltpu.VMEM),
                  pl.BlockSpec(memory_space=pltpu.SMEM),
                  pl.BlockSpec(memory_space=pltpu.SMEM)],
        scratch_shapes=[pltpu.VMEM((hp, c, wp), _F32)],
    )(zp, w1, w2, idx, gate)


def _add4_kernel(v_ref, g_ref, i_ref, l_ref, o_ref, *, hp):
    cpad = o_ref.shape[1] - v_ref.shape[1]
    wp = v_ref.shape[2]

    def row(y, carry):
        s = v_ref[y] + g_ref[y] + (i_ref[y] + l_ref[y])
        o_ref[y] = jnp.concatenate([s, jnp.zeros((cpad, wp), _F32)], axis=0)
        return carry

    jax.lax.fori_loop(0, hp, row, 0)


def _add4(vp, gp, ip, lp):
    hp, _, wp = vp.shape
    return pl.pallas_call(
        functools.partial(_add4_kernel, hp=hp),
        out_shape=jax.ShapeDtypeStruct((hp, 8, wp), _F32),
    )(vp, gp, ip, lp)


def _prep_w3(w, cin_pad=None):
    cout, cin = w.shape[0], w.shape[1]
    if cin_pad is not None and cin_pad > cin:
        w = jnp.pad(w, ((0, 0), (0, cin_pad - cin), (0, 0), (0, 0)))
        cin = cin_pad
    return jnp.transpose(w, (3, 0, 2, 1)).reshape(3, cout, 3 * cin)


def _prep_wexp(w):
    e, cout, cin = w.shape[0], w.shape[1], w.shape[2]
    return jnp.transpose(w, (0, 4, 1, 3, 2)).reshape(e, 3, cout, 3 * cin)


def _to_hcw_pad(x, cpad=None):
    t = jnp.transpose(x[0], (1, 0, 2))
    extra = 0 if cpad is None else max(0, cpad - t.shape[1])
    return jnp.pad(t, ((1, 1), (0, extra), (1, 1)))


def _to_hcw(x):
    return jnp.transpose(x[0], (1, 0, 2))


def _from_hcw(x):
    return jnp.transpose(x, (1, 0, 2))[None]


def _unpad(xp):
    return xp[1:-1, :, 1:-1]


def _branch(x_nchw, eps_nchw, w_stem, w_blocks, w_mu, w_sig, w_router,
            b_router, w_exp1, w_exp2):
    xp = _to_hcw_pad(x_nchw, cpad=8)
    h = _conv3x3(xp, _prep_w3(w_stem, cin_pad=8), relu=True, pad_out=True)
    for bi in range(w_blocks.shape[0]):
        h = _conv3x3(h, _prep_w3(w_blocks[bi]), relu=True, pad_out=True)

    wms = jnp.concatenate([w_mu[:, :, 0, 0], w_sig[:, :, 0, 0]], axis=0)
    mu, sg2 = _musig(h, wms)

    e = w_router.shape[0]
    wr = jnp.pad(w_router, ((0, 8 - e), (0, 0)))
    bmask = jnp.pad(b_router, (0, 8 - e),
                    constant_values=-1e30).reshape(8, 1).astype(_F32)
    zp, rt = _viz(mu, sg2, _to_hcw(eps_nchw), wr, bmask)

    idx = rt[0:2, 0].astype(jnp.int32)
    gate = rt[2:4, 0]
    dec = _moe(zp, _prep_wexp(w_exp1), _prep_wexp(w_exp2), idx, gate)
    return dec, mu, sg2


def kernel(i, v, eps_i, eps_v, W_ie_stem, W_ie_blocks, W_i_mu, W_i_sig,
           W_i_router, b_i_router, W_i_exp1, W_i_exp2, W_ve_stem, W_ve_blocks,
           W_v_mu, W_v_sig, W_v_router, b_v_router, W_v_exp1, W_v_exp2,
           W_f_stem, W_f_blocks, W_f_out):
    lp, mu_l, sg_l = _branch(i, eps_i, W_ie_stem, W_ie_blocks, W_i_mu, W_i_sig,
                             W_i_router, b_i_router, W_i_exp1, W_i_exp2)
    gp, mu_g, sg_g = _branch(v, eps_v, W_ve_stem, W_ve_blocks, W_v_mu, W_v_sig,
                             W_v_router, b_v_router, W_v_exp1, W_v_exp2)

    fused = _add4(_to_hcw_pad(v), gp, _to_hcw_pad(i), lp)
    fh = _conv3x3(fused, _prep_w3(W_f_stem, cin_pad=8), relu=True,
                  pad_out=True)
    for bi in range(W_f_blocks.shape[0]):
        fh = _conv3x3(fh, _prep_w3(W_f_blocks[bi]), relu=True, pad_out=True)
    fusion = _conv3x3(fh, _prep_w3(W_f_out), relu=False, pad_out=False)

    return (_from_hcw(fusion), _from_hcw(_unpad(lp)), _from_hcw(_unpad(gp)),
            _from_hcw(mu_l), _from_hcw(sg_l), _from_hcw(mu_g), _from_hcw(sg_g))


# speedup vs baseline: 3.6362x; 1.1099x over previous
import functools

import jax
import jax.numpy as jnp
from jax.experimental import pallas as pl
from jax.experimental.pallas import tpu as pltpu

_PREC = jax.lax.Precision.HIGHEST
_CPREC = jax.lax.Precision.DEFAULT
_F32 = jnp.float32


def _row3(x_ref, y, cin, wp):
    return x_ref[pl.ds(y, 3)].reshape(3 * cin, wp)


def _tap_matmul(wfun, x3, w):
    acc = jnp.dot(wfun(0), x3[:, 0:w], precision=_CPREC)
    acc = acc + jnp.dot(wfun(1), x3[:, 1:w + 1], precision=_CPREC)
    acc = acc + jnp.dot(wfun(2), x3[:, 2:w + 2], precision=_CPREC)
    return acc


def _pad_row(acc):
    cout = acc.shape[0]
    zcol = jnp.zeros((cout, 1), _F32)
    return jnp.concatenate([zcol, acc, zcol], axis=1)


_UNROLL = 8


def _conv_kernel(x_ref, w_ref, o_ref, *, hh, ww, relu, pad_out):
    cin = x_ref.shape[1]
    cout = o_ref.shape[1]
    if pad_out:
        o_ref[0] = jnp.zeros((cout, ww + 2), _F32)
        o_ref[hh + 1] = jnp.zeros((cout, ww + 2), _F32)

    def rows(it, carry):
        y0 = it * _UNROLL
        for u in range(_UNROLL):
            y = y0 + u
            x3 = _row3(x_ref, y, cin, ww + 2)
            acc = _tap_matmul(lambda dx: w_ref[dx], x3, ww)
            if relu:
                acc = jnp.maximum(acc, 0.0)
            if pad_out:
                o_ref[y + 1] = _pad_row(acc)
            else:
                o_ref[y] = acc
        return carry

    jax.lax.fori_loop(0, hh // _UNROLL, rows, 0)


def _conv3x3(xp, w3, relu, pad_out):
    hp, _, wp = xp.shape
    hh, ww = hp - 2, wp - 2
    cout = w3.shape[1]
    oshape = (hp, cout, wp) if pad_out else (hh, cout, ww)
    return pl.pallas_call(
        functools.partial(_conv_kernel, hh=hh, ww=ww, relu=relu,
                          pad_out=pad_out),
        out_shape=jax.ShapeDtypeStruct(oshape, _F32),
    )(xp, w3)


def _musig_kernel(h_ref, wms_ref, mu_ref, sg_ref, *, hh, ww):
    c = mu_ref.shape[1]

    def rows(it, carry):
        y0 = it * _UNROLL
        for u in range(_UNROLL):
            y = y0 + u
            hrow = h_ref[y + 1, :, 1:ww + 1]
            ms = jnp.dot(wms_ref[...], hrow, precision=_CPREC)
            mu_ref[y] = ms[:c]
            raw = ms[c:]
            sp = jnp.maximum(raw, 0.0) + jnp.log1p(jnp.exp(-jnp.abs(raw)))
            sg_ref[y] = sp + 1e-6
        return carry

    jax.lax.fori_loop(0, hh // _UNROLL, rows, 0)


def _musig(hp, wms):
    hp_, c2 = hp.shape[0], wms.shape[0]
    hh, ww = hp_ - 2, hp.shape[2] - 2
    c = c2 // 2
    return pl.pallas_call(
        functools.partial(_musig_kernel, hh=hh, ww=ww),
        out_shape=(jax.ShapeDtypeStruct((hh, c, ww), _F32),
                   jax.ShapeDtypeStruct((hh, c, ww), _F32)),
    )(hp, wms)


def _viz_kernel(mu_ref, sg_ref, eps_ref, wr_ref, bm_ref, z_ref, rt_ref,
                acc_ref, *, hh, ww):
    c = mu_ref.shape[1]
    z_ref[0] = jnp.zeros((c, ww + 2), _F32)
    z_ref[hh + 1] = jnp.zeros((c, ww + 2), _F32)
    acc_ref[...] = jnp.zeros((c, ww), _F32)

    def rows(it, carry):
        y0 = it * _UNROLL
        acc = acc_ref[...]
        for u in range(_UNROLL):
            y = y0 + u
            z = mu_ref[y] + jnp.sqrt(sg_ref[y]) * eps_ref[y]
            z_ref[y + 1] = _pad_row(z)
            acc = acc + z
        acc_ref[...] = acc
        return carry

    jax.lax.fori_loop(0, hh // _UNROLL, rows, 0)

    pooled_mat = jnp.dot(wr_ref[...], acc_ref[...], precision=_PREC)
    logits = jnp.sum(pooled_mat, axis=1, keepdims=True) / (hh * ww)
    logits = logits + bm_ref[...]
    sub = jax.lax.broadcasted_iota(jnp.int32, (8, 1), 0)
    neg = jnp.float32(-1e30)
    m1 = jnp.max(logits)
    i1 = -jnp.max(jnp.where(logits == m1, -sub.astype(_F32), neg))
    masked = jnp.where(sub.astype(_F32) == i1, neg, logits)
    m2 = jnp.max(masked)
    i2 = -jnp.max(jnp.where(masked == m2, -sub.astype(_F32), neg))
    e1 = jnp.exp(m1 - m1)
    e2 = jnp.exp(m2 - m1)
    g1 = e1 / (e1 + e2)
    g2 = e2 / (e1 + e2)
    out = jnp.where(sub == 0, i1,
          jnp.where(sub == 1, i2,
          jnp.where(sub == 2, g1,
          jnp.where(sub == 3, g2, 0.0))))
    rt_ref[...] = jnp.broadcast_to(out, (8, 128))


def _viz(mu, sg, eps, wr, bmask):
    hh, c, ww = mu.shape
    return pl.pallas_call(
        functools.partial(_viz_kernel, hh=hh, ww=ww),
        out_shape=(jax.ShapeDtypeStruct((hh + 2, c, ww + 2), _F32),
                   jax.ShapeDtypeStruct((8, 128), _F32)),
        scratch_shapes=[pltpu.VMEM((c, ww), _F32)],
    )(mu, sg, eps, wr, bmask)


def _moe_kernel(z_ref, w1_ref, w2_ref, idx_ref, gate_ref, o_ref, a_ref,
                *, hh, ww):
    c = z_ref.shape[1]
    cout = o_ref.shape[1]
    o_ref[0] = jnp.zeros((cout, ww + 2), _F32)
    o_ref[hh + 1] = jnp.zeros((cout, ww + 2), _F32)
    a_ref[0] = jnp.zeros((c, ww + 2), _F32)
    a_ref[hh + 1] = jnp.zeros((c, ww + 2), _F32)

    for k in (0, 1):
        e = idx_ref[k]
        g = gate_ref[k]

        def rows1(it, carry):
            y0 = it * _UNROLL
            for u in range(_UNROLL):
                y = y0 + u
                x3 = _row3(z_ref, y, c, ww + 2)
                acc = _tap_matmul(lambda dx: w1_ref[e, dx], x3, ww)
                a_ref[y + 1] = _pad_row(jnp.maximum(acc, 0.0))
            return carry

        jax.lax.fori_loop(0, hh // _UNROLL, rows1, 0)

        def rows2(it, carry):
            y0 = it * _UNROLL
            for u in range(_UNROLL):
                y = y0 + u
                x3 = _row3(a_ref, y, c, ww + 2)
                acc = _tap_matmul(lambda dx: w2_ref[e, dx], x3, ww) * g
                if k == 0:
                    o_ref[y + 1] = _pad_row(acc)
                else:
                    o_ref[y + 1] = o_ref[y + 1] + _pad_row(acc)
            return carry

        jax.lax.fori_loop(0, hh // _UNROLL, rows2, 0)


def _moe(zp, w1, w2, idx, gate):
    hp, c, wp = zp.shape
    hh, ww = hp - 2, wp - 2
    cout = w2.shape[2]
    return pl.pallas_call(
        functools.partial(_moe_kernel, hh=hh, ww=ww),
        out_shape=jax.ShapeDtypeStruct((hp, cout, wp), _F32),
        in_specs=[pl.BlockSpec(memory_space=pltpu.VMEM),
                  pl.BlockSpec(memory_space=pltpu.VMEM),
                  pl.BlockSpec(memory_space=pltpu.VMEM),
                  pl.BlockSpec(memory_space=pltpu.SMEM),
                  pl.BlockSpec(memory_space=pltpu.SMEM)],
        scratch_shapes=[pltpu.VMEM((hp, c, wp), _F32)],
    )(zp, w1, w2, idx, gate)


def _add4_kernel(v_ref, g_ref, i_ref, l_ref, o_ref, *, hp):
    cpad = o_ref.shape[1] - v_ref.shape[1]
    wp = v_ref.shape[2]

    def rows(it, carry):
        y0 = it * 2
        for u in range(2):
            y = y0 + u
            s = v_ref[y] + g_ref[y] + (i_ref[y] + l_ref[y])
            o_ref[y] = jnp.concatenate([s, jnp.zeros((cpad, wp), _F32)],
                                       axis=0)
        return carry

    jax.lax.fori_loop(0, hp // 2, rows, 0)


def _add4(vp, gp, ip, lp):
    hp, _, wp = vp.shape
    return pl.pallas_call(
        functools.partial(_add4_kernel, hp=hp),
        out_shape=jax.ShapeDtypeStruct((hp, 8, wp), _F32),
    )(vp, gp, ip, lp)


def _prep_w3(w, cin_pad=None):
    cout, cin = w.shape[0], w.shape[1]
    if cin_pad is not None and cin_pad > cin:
        w = jnp.pad(w, ((0, 0), (0, cin_pad - cin), (0, 0), (0, 0)))
        cin = cin_pad
    return jnp.transpose(w, (3, 0, 2, 1)).reshape(3, cout, 3 * cin)


def _prep_wexp(w):
    e, cout, cin = w.shape[0], w.shape[1], w.shape[2]
    return jnp.transpose(w, (0, 4, 1, 3, 2)).reshape(e, 3, cout, 3 * cin)


def _to_hcw_pad(x, cpad=None):
    t = jnp.transpose(x[0], (1, 0, 2))
    extra = 0 if cpad is None else max(0, cpad - t.shape[1])
    return jnp.pad(t, ((1, 1), (0, extra), (1, 1)))


def _to_hcw(x):
    return jnp.transpose(x[0], (1, 0, 2))


def _from_hcw(x):
    return jnp.transpose(x, (1, 0, 2))[None]


def _unpad(xp):
    return xp[1:-1, :, 1:-1]


def _branch(x_nchw, eps_nchw, w_stem, w_blocks, w_mu, w_sig, w_router,
            b_router, w_exp1, w_exp2):
    xp = _to_hcw_pad(x_nchw, cpad=8)
    h = _conv3x3(xp, _prep_w3(w_stem, cin_pad=8), relu=True, pad_out=True)
    for bi in range(w_blocks.shape[0]):
        h = _conv3x3(h, _prep_w3(w_blocks[bi]), relu=True, pad_out=True)

    wms = jnp.concatenate([w_mu[:, :, 0, 0], w_sig[:, :, 0, 0]], axis=0)
    mu, sg2 = _musig(h, wms)

    e = w_router.shape[0]
    wr = jnp.pad(w_router, ((0, 8 - e), (0, 0)))
    bmask = jnp.pad(b_router, (0, 8 - e),
                    constant_values=-1e30).reshape(8, 1).astype(_F32)
    zp, rt = _viz(mu, sg2, _to_hcw(eps_nchw), wr, bmask)

    idx = rt[0:2, 0].astype(jnp.int32)
    gate = rt[2:4, 0]
    dec = _moe(zp, _prep_wexp(w_exp1), _prep_wexp(w_exp2), idx, gate)
    return dec, mu, sg2


def kernel(i, v, eps_i, eps_v, W_ie_stem, W_ie_blocks, W_i_mu, W_i_sig,
           W_i_router, b_i_router, W_i_exp1, W_i_exp2, W_ve_stem, W_ve_blocks,
           W_v_mu, W_v_sig, W_v_router, b_v_router, W_v_exp1, W_v_exp2,
           W_f_stem, W_f_blocks, W_f_out):
    lp, mu_l, sg_l = _branch(i, eps_i, W_ie_stem, W_ie_blocks, W_i_mu, W_i_sig,
                             W_i_router, b_i_router, W_i_exp1, W_i_exp2)
    gp, mu_g, sg_g = _branch(v, eps_v, W_ve_stem, W_ve_blocks, W_v_mu, W_v_sig,
                             W_v_router, b_v_router, W_v_exp1, W_v_exp2)

    fused = _add4(_to_hcw_pad(v), gp, _to_hcw_pad(i), lp)
    fh = _conv3x3(fused, _prep_w3(W_f_stem, cin_pad=8), relu=True,
                  pad_out=True)
    for bi in range(W_f_blocks.shape[0]):
        fh = _conv3x3(fh, _prep_w3(W_f_blocks[bi]), relu=True, pad_out=True)
    fusion = _conv3x3(fh, _prep_w3(W_f_out), relu=False, pad_out=False)

    return (_from_hcw(fusion), _from_hcw(_unpad(lp)), _from_hcw(_unpad(gp)),
            _from_hcw(mu_l), _from_hcw(sg_l), _from_hcw(mu_g), _from_hcw(sg_g))
